# baseline (speedup 1.0000x reference)
import functools

import jax
import jax.numpy as jnp
from jax import lax
from jax.experimental import pallas as pl
from jax.experimental.pallas import tpu as pltpu
from jax.experimental.pallas import tpu_sc as plsc

NUM_TABLES = 9
VOCAB = 100
HIDDEN = 64
HPAIR = HIDDEN // 2
E = 800000

_info = plsc.get_sparse_core_info()
NC, NS, L = _info.num_cores, _info.num_subcores, _info.num_lanes
NW = NC * NS
EPW = E // NW
CHUNK = 192
NCHUNKS = -(-EPW // CHUNK)
NB = 2


def _sc_body(edge_hbm, tab_hbm, out_hbm, tab_v, idx_v0, idx_v1, out_v0,
             out_v1, tab_sem, idx_sem0, idx_sem1, out_sem0, out_sem1):
    idx_vs = [idx_v0, idx_v1]
    out_vs = [out_v0, out_v1]
    idx_sems = [idx_sem0, idx_sem1]
    out_sems = [out_sem0, out_sem1]
    wid = lax.axis_index("s") * NC + lax.axis_index("c")
    base0 = wid * EPW
    tab_cp = pltpu.make_async_copy(tab_hbm, tab_v, tab_sem)
    tab_cp.start()

    offc = lax.iota(jnp.int32, L) * (VOCAB * HPAIR)
    lanecol = lax.iota(jnp.int32, L)
    colmask = lanecol < NUM_TABLES
    zeros = jnp.zeros((L,), jnp.int32)

    def chunk_base(kc):
        if isinstance(kc, int):
            return base0 + min(kc * CHUNK, EPW - CHUNK)
        return base0 + jnp.minimum(kc * CHUNK, EPW - CHUNK)

    def idx_copy(kc, b):
        return pltpu.make_async_copy(
            edge_hbm.at[pl.ds(chunk_base(kc), CHUNK)],
            idx_vs[b],
            idx_sems[b],
        )

    def out_copy(kc, b):
        return pltpu.make_async_copy(
            out_vs[b],
            out_hbm.at[pl.ds(chunk_base(kc), CHUNK)],
            out_sems[b],
        )

    for b in range(NB):
        idx_copy(b, b).start()
    tab_cp.wait()

    def process(kc, b):
        idx_copy(kc, b).wait()
        @pl.when(kc >= NB)
        def _():
            out_copy(kc - NB, b).wait()

        @plsc.parallel_loop(0, CHUNK, unroll=8)
        def edge_body(e):
            iv = plsc.load_gather(idx_vs[b], [zeros + e, lanecol],
                                  mask=colmask)
            av = iv * HPAIR + offc
            accs = [None] * (HPAIR // L)
            for i in range(NUM_TABLES):
                off = av[i]
                for j in range(HPAIR // L):
                    v = plsc.bitcast(tab_v[pl.ds(off + j * L, L)],
                                     jnp.bfloat16)
                    accs[j] = v if accs[j] is None else accs[j] + v
            for j in range(HPAIR // L):
                a, c = plsc.unpack(accs[j], format=plsc.PackFormat.INTERLEAVED,
                                   preferred_element_type=jnp.float32)
                out_vs[b][e, pl.ds(j * 2 * L, L)] = a
                out_vs[b][e, pl.ds((j * 2 + 1) * L, L)] = c

        out_copy(kc, b).start()
        @pl.when(kc + NB < NCHUNKS)
        def _():
            idx_copy(kc + NB, b).start()

    def chunk_group(kk, _):
        for b in range(NB):
            process(kk * NB + b, b)
        return 0

    lax.fori_loop(0, NCHUNKS // NB, chunk_group, 0)
    for b in range(NCHUNKS % NB):
        process((NCHUNKS // NB) * NB + b, b)
    for kc in range(NCHUNKS - NB, NCHUNKS):
        out_copy(kc, kc % NB).wait()


@jax.jit
def _encode(edge_attr, tab_packed):
    mesh = plsc.VectorSubcoreMesh(core_axis_name="c", subcore_axis_name="s")
    run = pl.kernel(
        _sc_body,
        out_type=jax.ShapeDtypeStruct((E, HIDDEN), jnp.float32),
        mesh=mesh,
        scratch_types=[
            pltpu.VMEM((NUM_TABLES * VOCAB * HPAIR,), jnp.int32),
            pltpu.VMEM((CHUNK, NUM_TABLES), jnp.int32),
            pltpu.VMEM((CHUNK, NUM_TABLES), jnp.int32),
            pltpu.VMEM((CHUNK, HIDDEN), jnp.float32),
            pltpu.VMEM((CHUNK, HIDDEN), jnp.float32),
            pltpu.SemaphoreType.DMA,
            pltpu.SemaphoreType.DMA,
            pltpu.SemaphoreType.DMA,
            pltpu.SemaphoreType.DMA,
            pltpu.SemaphoreType.DMA,
        ],
        compiler_params=pltpu.CompilerParams(needs_layout_passes=False),
    )
    return run(edge_attr, tab_packed)


def kernel(edge_attr, tables):
    edge_attr = edge_attr.astype(jnp.int32)
    t = tables.astype(jnp.bfloat16).reshape(NUM_TABLES * VOCAB, 2, 2, L)
    t = t.transpose(0, 1, 3, 2).reshape(NUM_TABLES * VOCAB * HPAIR, 2)
    tab_packed = lax.bitcast_convert_type(t, jnp.int32)
    return _encode(edge_attr, tab_packed)


# speedup vs baseline: 1.0171x; 1.0171x over previous
import functools

import jax
import jax.numpy as jnp
from jax import lax
from jax.experimental import pallas as pl
from jax.experimental.pallas import tpu as pltpu
from jax.experimental.pallas import tpu_sc as plsc

NUM_TABLES = 9
VOCAB = 100
HIDDEN = 64
HPAIR = HIDDEN // 2
E = 800000

_info = plsc.get_sparse_core_info()
NC, NS, L = _info.num_cores, _info.num_subcores, _info.num_lanes
NW = NC * NS
EPW = E // NW
CHUNK = 192
NCHUNKS = -(-EPW // CHUNK)
NB = 2


def _sc_body(edge_hbm, tab_hbm, out_hbm, tab_v, idx_v0, idx_v1, out_v0,
             out_v1, tab_sem, idx_sem0, idx_sem1, out_sem0, out_sem1):
    idx_vs = [idx_v0, idx_v1]
    out_vs = [out_v0, out_v1]
    idx_sems = [idx_sem0, idx_sem1]
    out_sems = [out_sem0, out_sem1]
    wid = lax.axis_index("s") * NC + lax.axis_index("c")
    base0 = wid * EPW
    tab_cp = pltpu.make_async_copy(tab_hbm, tab_v, tab_sem)
    tab_cp.start()

    offc = lax.iota(jnp.int32, L) * (VOCAB * HPAIR)
    lanecol = lax.iota(jnp.int32, L)
    colmask = lanecol < NUM_TABLES
    zeros = jnp.zeros((L,), jnp.int32)

    def chunk_base(kc):
        if isinstance(kc, int):
            return base0 + min(kc * CHUNK, EPW - CHUNK)
        return base0 + jnp.minimum(kc * CHUNK, EPW - CHUNK)

    def idx_copy(kc, b):
        return pltpu.make_async_copy(
            edge_hbm.at[pl.ds(chunk_base(kc), CHUNK)],
            idx_vs[b],
            idx_sems[b],
        )

    def out_copy(kc, b):
        return pltpu.make_async_copy(
            out_vs[b],
            out_hbm.at[pl.ds(chunk_base(kc), CHUNK)],
            out_sems[b],
        )

    for b in range(NB):
        idx_copy(b, b).start()
    tab_cp.wait()

    def process(kc, b):
        idx_copy(kc, b).wait()
        @pl.when(kc >= NB)
        def _():
            out_copy(kc - NB, b).wait()

        @plsc.parallel_loop(0, CHUNK, unroll=8)
        def edge_body(e):
            iv = plsc.load_gather(idx_vs[b], [zeros + e, lanecol],
                                  mask=colmask)
            av = iv * HPAIR + offc
            offs = [av[i] for i in range(NUM_TABLES)]
            accs = [None] * (HPAIR // L)
            for j in range(HPAIR // L):
                vs = [plsc.bitcast(tab_v[pl.ds(off + j * L, L)],
                                   jnp.bfloat16) for off in offs]
                while len(vs) > 1:
                    nxt = [vs[k] + vs[k + 1] for k in range(0, len(vs) - 1, 2)]
                    if len(vs) % 2:
                        nxt.append(vs[-1])
                    vs = nxt
                accs[j] = vs[0]
            for j in range(HPAIR // L):
                a, c = plsc.unpack(accs[j], format=plsc.PackFormat.INTERLEAVED,
                                   preferred_element_type=jnp.float32)
                out_vs[b][e, pl.ds(j * 2 * L, L)] = a
                out_vs[b][e, pl.ds((j * 2 + 1) * L, L)] = c

        out_copy(kc, b).start()
        @pl.when(kc + NB < NCHUNKS)
        def _():
            idx_copy(kc + NB, b).start()

    def chunk_group(kk, _):
        for b in range(NB):
            process(kk * NB + b, b)
        return 0

    lax.fori_loop(0, NCHUNKS // NB, chunk_group, 0)
    for b in range(NCHUNKS % NB):
        process((NCHUNKS // NB) * NB + b, b)
    for kc in range(NCHUNKS - NB, NCHUNKS):
        out_copy(kc, kc % NB).wait()


@jax.jit
def _encode(edge_attr, tab_packed):
    mesh = plsc.VectorSubcoreMesh(core_axis_name="c", subcore_axis_name="s")
    run = pl.kernel(
        _sc_body,
        out_type=jax.ShapeDtypeStruct((E, HIDDEN), jnp.float32),
        mesh=mesh,
        scratch_types=[
            pltpu.VMEM((NUM_TABLES * VOCAB * HPAIR,), jnp.int32),
            pltpu.VMEM((CHUNK, NUM_TABLES), jnp.int32),
            pltpu.VMEM((CHUNK, NUM_TABLES), jnp.int32),
            pltpu.VMEM((CHUNK, HIDDEN), jnp.float32),
            pltpu.VMEM((CHUNK, HIDDEN), jnp.float32),
            pltpu.SemaphoreType.DMA,
            pltpu.SemaphoreType.DMA,
            pltpu.SemaphoreType.DMA,
            pltpu.SemaphoreType.DMA,
            pltpu.SemaphoreType.DMA,
        ],
        compiler_params=pltpu.CompilerParams(needs_layout_passes=False),
    )
    return run(edge_attr, tab_packed)


def kernel(edge_attr, tables):
    edge_attr = edge_attr.astype(jnp.int32)
    t = tables.astype(jnp.bfloat16).reshape(NUM_TABLES * VOCAB, 2, 2, L)
    t = t.transpose(0, 1, 3, 2).reshape(NUM_TABLES * VOCAB * HPAIR, 2)
    tab_packed = lax.bitcast_convert_type(t, jnp.int32)
    return _encode(edge_attr, tab_packed)
